# Initial kernel scaffold; baseline (speedup 1.0000x reference)
import functools

import jax
import jax.numpy as jnp
from jax import lax
from jax.experimental import pallas as pl
from jax.experimental.pallas import tpu as pltpu
from jax.experimental.pallas import tpu_sc as plsc

N = 10000
D = 128
E = 320000

NC = 2
NS = 16
NW = NC * NS

K = 128
CH = 79
EW = CH * K
E_PAD = NW * EW

N_PAD = 10112
RW = N_PAD // NS
DUMMY = N

_sc_mesh = plsc.VectorSubcoreMesh(
    core_axis_name="c", subcore_axis_name="s", num_cores=NC, num_subcores=NS
)


def _sc_agg_body(x_hbm, src_hbm, dst_hbm, zrows_hbm, zdeg_hbm,
                 acc_out, deg_out,
                 acc_sh, deg_sh, src_v, dst_v, rows_v, ones_v, sem):
    cid = lax.axis_index("c")
    sid = lax.axis_index("s")
    wid = cid * NS + sid

    pltpu.sync_copy(zrows_hbm.at[pl.ds(sid * RW, RW)],
                    acc_sh.at[pl.ds(sid * RW, RW)])
    pltpu.sync_copy(zdeg_hbm.at[pl.ds(sid * RW, RW)],
                    deg_sh.at[pl.ds(sid * RW, RW)])

    pltpu.sync_copy(src_hbm.at[wid], src_v)
    pltpu.sync_copy(dst_hbm.at[wid], dst_v)

    for j in range(K // 16):
        ones_v[pl.ds(j * 16, 16)] = jnp.ones((16,), jnp.float32)

    plsc.subcore_barrier()

    def chunk(c, carry):
        pltpu.async_copy(x_hbm.at[src_v.at[c]], rows_v, sem).wait()
        pltpu.sync_copy(rows_v, acc_sh.at[dst_v.at[c]], add=True)
        pltpu.sync_copy(ones_v, deg_sh.at[dst_v.at[c]], add=True)
        return carry

    lax.fori_loop(0, CH, chunk, 0)

    plsc.subcore_barrier()

    pltpu.sync_copy(acc_sh.at[pl.ds(sid * RW, RW)],
                    acc_out.at[cid, pl.ds(sid * RW, RW)])
    pltpu.sync_copy(deg_sh.at[pl.ds(sid * RW, RW)],
                    deg_out.at[cid, pl.ds(sid * RW, RW)])


_sc_agg = pl.kernel(
    _sc_agg_body,
    out_type=[
        jax.ShapeDtypeStruct((NC, N_PAD, D), jnp.float32),
        jax.ShapeDtypeStruct((NC, N_PAD), jnp.float32),
    ],
    mesh=_sc_mesh,
    scratch_types=[
        pltpu.VMEM_SHARED((N_PAD, D), jnp.float32),
        pltpu.VMEM_SHARED((N_PAD,), jnp.float32),
        pltpu.VMEM((CH, K), jnp.int32),
        pltpu.VMEM((CH, K), jnp.int32),
        pltpu.VMEM((K, D), jnp.float32),
        pltpu.VMEM((K,), jnp.float32),
        pltpu.SemaphoreType.DMA,
    ],
)


_TC_R = 1000


def _tc_dense_body(acc_ref, deg_ref, h_ref, wn_ref, ws_ref, b_ref, out_ref):
    p = acc_ref[0] + acc_ref[1]
    d = jnp.maximum(deg_ref[0] + deg_ref[1], 1.0)
    agg = p / d
    y = (jnp.dot(agg, wn_ref[...], preferred_element_type=jnp.float32,
                 precision=lax.Precision.HIGHEST)
         + jnp.dot(h_ref[...], ws_ref[...], preferred_element_type=jnp.float32,
                   precision=lax.Precision.HIGHEST)
         + b_ref[...])
    out_ref[...] = jnp.maximum(y, 0.0)


def _tc_dense(acc, deg, h, w_neigh, w_self, b):
    return pl.pallas_call(
        _tc_dense_body,
        grid=(N // _TC_R,),
        in_specs=[
            pl.BlockSpec((NC, _TC_R, D), lambda i: (0, i, 0)),
            pl.BlockSpec((NC, _TC_R, 1), lambda i: (0, i, 0)),
            pl.BlockSpec((_TC_R, D), lambda i: (i, 0)),
            pl.BlockSpec((D, D), lambda i: (0, 0)),
            pl.BlockSpec((D, D), lambda i: (0, 0)),
            pl.BlockSpec((1, D), lambda i: (0, 0)),
        ],
        out_specs=pl.BlockSpec((_TC_R, D), lambda i: (i, 0)),
        out_shape=jax.ShapeDtypeStruct((N, D), jnp.float32),
    )(acc, deg, h, w_neigh, w_self, b)


def kernel(x, edge_index, W_self1, W_neigh1, b1, W_self2, W_neigh2, b2):
    e = edge_index.astype(jnp.int32)
    pad = E_PAD - E
    src = jnp.concatenate([e[0], jnp.zeros((pad,), jnp.int32)]).reshape(NW, CH, K)
    dst = jnp.concatenate([e[1], jnp.full((pad,), DUMMY, jnp.int32)]).reshape(NW, CH, K)
    zrows = jnp.zeros((N_PAD, D), jnp.float32)
    zdeg = jnp.zeros((N_PAD,), jnp.float32)
    b1r = b1.reshape(1, D)
    b2r = b2.reshape(1, D)

    acc1, deg = _sc_agg(x, src, dst, zrows, zdeg)
    deg3 = deg[:, :, None]
    h1 = _tc_dense(acc1, deg3, x, W_neigh1, W_self1, b1r)
    acc2, _ = _sc_agg(h1, src, dst, zrows, zdeg)
    h2 = _tc_dense(acc2, deg3, h1, W_neigh2, W_self2, b2r)
    return h2


# speedup vs baseline: 5.3937x; 5.3937x over previous
import functools

import jax
import jax.numpy as jnp
from jax import lax
from jax.experimental import pallas as pl
from jax.experimental.pallas import tpu as pltpu
from jax.experimental.pallas import tpu_sc as plsc

N = 10000
D = 128
E = 320000

NC = 2
NS = 16
NW = NC * NS

K = 128
CH = 79
EW = CH * K
E_PAD = NW * EW

N_PAD = 10112
RW = N_PAD // NS
DUMMY = N

_sc_mesh = plsc.VectorSubcoreMesh(
    core_axis_name="c", subcore_axis_name="s", num_cores=NC, num_subcores=NS
)


def _sc_agg_body(x_hbm, src_hbm, dst_hbm, zrows_hbm, zdeg_hbm,
                 acc_out, deg_out,
                 acc_sh, deg_sh, src_v, dst_v, rows_v, ones_v, deg_v, sem):
    cid = lax.axis_index("c")
    sid = lax.axis_index("s")
    wid = cid * NS + sid

    pltpu.sync_copy(zrows_hbm.at[pl.ds(sid * RW, RW)],
                    acc_sh.at[pl.ds(sid * RW, RW)])
    pltpu.sync_copy(zdeg_hbm.at[pl.ds(sid * RW, RW)], deg_v)
    pltpu.sync_copy(deg_v, deg_sh.at[pl.ds(sid * RW, RW)])

    pltpu.sync_copy(src_hbm.at[wid], src_v)
    pltpu.sync_copy(dst_hbm.at[wid], dst_v)

    for j in range(K // 16):
        ones_v[pl.ds(j * 16, 16)] = jnp.ones((16,), jnp.float32)

    plsc.subcore_barrier()

    def chunk(c, carry):
        pltpu.async_copy(x_hbm.at[src_v.at[c]], rows_v, sem).wait()
        pltpu.sync_copy(rows_v, acc_sh.at[dst_v.at[c]], add=True)
        pltpu.sync_copy(ones_v, deg_sh.at[dst_v.at[c]], add=True)
        return carry

    lax.fori_loop(0, CH, chunk, 0)

    plsc.subcore_barrier()

    pltpu.sync_copy(acc_sh.at[pl.ds(sid * RW, RW)],
                    acc_out.at[cid, pl.ds(sid * RW, RW)])
    pltpu.sync_copy(deg_sh.at[pl.ds(sid * RW, RW)], deg_v)
    pltpu.sync_copy(deg_v, deg_out.at[pl.ds(cid * N_PAD + sid * RW, RW)])


_sc_agg = pl.kernel(
    _sc_agg_body,
    out_type=[
        jax.ShapeDtypeStruct((NC, N_PAD, D), jnp.float32),
        jax.ShapeDtypeStruct((NC * N_PAD,), jnp.float32),
    ],
    mesh=_sc_mesh,
    scratch_types=[
        pltpu.VMEM_SHARED((N_PAD, D), jnp.float32),
        pltpu.VMEM_SHARED((N_PAD,), jnp.float32),
        pltpu.VMEM((CH, K), jnp.int32),
        pltpu.VMEM((CH, K), jnp.int32),
        pltpu.VMEM((K, D), jnp.float32),
        pltpu.VMEM((K,), jnp.float32),
        pltpu.VMEM((RW,), jnp.float32),
        pltpu.SemaphoreType.DMA,
    ],
)


_TC_R = 1000


def _tc_dense_body(acc_ref, deg_ref, h_ref, wn_ref, ws_ref, b_ref, out_ref):
    p = acc_ref[0] + acc_ref[1]
    d = jnp.maximum(deg_ref[0] + deg_ref[1], 1.0)
    agg = p / d
    y = (jnp.dot(agg, wn_ref[...], preferred_element_type=jnp.float32,
                 precision=lax.Precision.HIGHEST)
         + jnp.dot(h_ref[...], ws_ref[...], preferred_element_type=jnp.float32,
                   precision=lax.Precision.HIGHEST)
         + b_ref[...])
    out_ref[...] = jnp.maximum(y, 0.0)


def _tc_dense(acc, deg, h, w_neigh, w_self, b):
    return pl.pallas_call(
        _tc_dense_body,
        grid=(N // _TC_R,),
        in_specs=[
            pl.BlockSpec((NC, _TC_R, D), lambda i: (0, i, 0)),
            pl.BlockSpec((NC, _TC_R, 1), lambda i: (0, i, 0)),
            pl.BlockSpec((_TC_R, D), lambda i: (i, 0)),
            pl.BlockSpec((D, D), lambda i: (0, 0)),
            pl.BlockSpec((D, D), lambda i: (0, 0)),
            pl.BlockSpec((1, D), lambda i: (0, 0)),
        ],
        out_specs=pl.BlockSpec((_TC_R, D), lambda i: (i, 0)),
        out_shape=jax.ShapeDtypeStruct((N, D), jnp.float32),
    )(acc, deg, h, w_neigh, w_self, b)


def kernel(x, edge_index, W_self1, W_neigh1, b1, W_self2, W_neigh2, b2):
    e = edge_index.astype(jnp.int32)
    pad = E_PAD - E
    src = jnp.concatenate([e[0], jnp.zeros((pad,), jnp.int32)]).reshape(NW, CH, K)
    dst = jnp.concatenate([e[1], jnp.full((pad,), DUMMY, jnp.int32)]).reshape(NW, CH, K)
    zrows = jnp.zeros((N_PAD, D), jnp.float32)
    zdeg = jnp.zeros((N_PAD,), jnp.float32)
    b1r = b1.reshape(1, D)
    b2r = b2.reshape(1, D)

    acc1, deg = _sc_agg(x, src, dst, zrows, zdeg)
    deg3 = deg.reshape(NC, N_PAD, 1)
    h1 = _tc_dense(acc1, deg3, x, W_neigh1, W_self1, b1r)
    acc2, _ = _sc_agg(h1, src, dst, zrows, zdeg)
    h2 = _tc_dense(acc2, deg3, h1, W_neigh2, W_self2, b2r)
    return h2
